# baseline (speedup 1.0000x reference)
import functools

import jax
import jax.numpy as jnp
from jax import lax
from jax.experimental import pallas as pl
from jax.experimental.pallas import tpu as pltpu
from jax.experimental.pallas import tpu_sc as plsc

B = 16
A = 4096
FD = 128
L = 16
NK = FD // L
CHUNK = 256
MOLC = A // CHUNK
SCB = 2
TCB = B - SCB
MPC = SCB // 2
NSUB = 16
MAXSLOT = max(1, MPC * MOLC // NSUB)
UNROLL = 8
PARTROWS = MPC * MOLC


def _sc_mol_kernel(nf_hbm, ds_hbm, out_hbm,
                   buf0, buf1, buf2, stage, rowp, comb, row_buf, shared,
                   sem0, sem1, sem2, sem_r):
    core = lax.axis_index("c")
    sub = lax.axis_index("s")
    idx16 = lax.iota(jnp.int32, L)

    pltpu.async_copy(ds_hbm, stage.at[pl.ds(0, 2 * B)], sem_r)
    pltpu.make_async_copy(ds_hbm, stage.at[pl.ds(0, 2 * B)], sem_r).wait()

    def va_of(j):
        return stage[pl.ds(MPC * core + j, L)][0]

    va_l = [va_of(j) for j in range(MPC)]
    cum = [jnp.int32(0)]
    for j in range(MPC):
        cum.append(cum[j] + (va_l[j] + CHUNK - 1) // CHUNK)
    total = cum[MPC]

    q = total // NSUB
    r = total % NSUB
    my_cnt = q + jnp.where(sub < r, 1, 0)
    my_start = sub * q + jnp.minimum(sub, r)

    def chunk_info(i):
        g = my_start + i
        lb = jnp.int32(0)
        ce = jnp.int32(0)
        va = va_l[0]
        for j in range(1, MPC):
            after = cum[j] <= g
            lb = lb + after.astype(jnp.int32)
            ce = jnp.where(after, cum[j], ce)
            va = jnp.where(after, va_l[j], va)
        jc = g - ce
        m = jnp.clip(va - jc * CHUNK, 0, CHUNK)
        return g, MPC * core + lb, jc, m

    bufs = [buf0, buf1, buf2]
    sems = [sem0, sem1, sem2]
    NBUF = 3

    def dma_start(i):
        g, b, jc, m = chunk_info(i)

        @pl.when(i < my_cnt)
        def _():
            pltpu.async_copy(
                nf_hbm.at[b, pl.ds(jc * CHUNK, CHUNK), :], bufs[i % NBUF],
                sems[i % NBUF])

    dma_start(0)
    if MAXSLOT > 1:
        dma_start(1)
    for i in range(MAXSLOT):
        if i + 2 < MAXSLOT:
            dma_start(i + 2)
        g, b, jc, m = chunk_info(i)
        buf = bufs[i % NBUF]

        @pl.when(i < my_cnt)
        def _(buf=buf, g=g, b=b, jc=jc, m=m, i=i):
            pltpu.make_async_copy(
                nf_hbm.at[b, pl.ds(jc * CHUNK, CHUNK), :], buf,
                sems[i % NBUF]).wait()
            accs = tuple(jnp.zeros((L,), jnp.float32) for _ in range(NK))

            def body(it, acc, buf=buf, m=m):
                base = it * UNROLL
                for rr in range(UNROLL):
                    j = base + rr
                    keep = j < m
                    acc = tuple(
                        acc[k] + jnp.where(keep, buf[j, pl.ds(k * L, L)],
                                           jnp.float32(0.0))
                        for k in range(NK))
                return acc

            ngroups = (m + UNROLL - 1) // UNROLL
            accs = lax.fori_loop(0, ngroups, body, accs)
            for k in range(NK):
                rowp[i, pl.ds(k * L, L)] = accs[k]
            pltpu.async_copy(rowp.at[i], shared.at[g], sem_r)

    for i in range(MAXSLOT):
        g, b, jc, m = chunk_info(i)

        @pl.when(i < my_cnt)
        def _(g=g, i=i):
            pltpu.make_async_copy(rowp.at[i], shared.at[g], sem_r).wait()
    plsc.subcore_barrier()

    @pl.when(sub < MPC)
    def _():
        b_out = MPC * core + sub
        cs = jnp.int32(0)
        n = jnp.int32(0)
        for j in range(MPC):
            here = sub == j
            cs = jnp.where(here, cum[j], cs)
            n = jnp.where(here, cum[j + 1] - cum[j], n)
        vf_b = stage[pl.ds(B + b_out, L)][0]
        pltpu.sync_copy(shared.at[pl.ds(cs, MOLC)], comb)
        accs = tuple(jnp.zeros((L,), jnp.float32) for _ in range(NK))
        for rr in range(MOLC):
            keep = rr < n
            accs = tuple(
                accs[k] + jnp.where(keep, comb[rr, pl.ds(k * L, L)],
                                    jnp.float32(0.0))
                for k in range(NK))
        for k in range(NK):
            fkeep = (idx16 + k * L) < vf_b
            row_buf[pl.ds(k * L, L)] = jnp.maximum(
                jnp.where(fkeep, accs[k], jnp.float32(0.0)), jnp.float32(0.0))
        pltpu.sync_copy(row_buf, out_hbm.at[b_out])


def _tc_mol_kernel(lc_ref, va_ref, vf_ref, nf_ref, out_ref):
    b = pl.program_id(0)
    j = pl.program_id(1)
    mb = b + SCB

    @pl.when(j == 0)
    def _():
        out_ref[...] = jnp.zeros_like(out_ref)

    va = va_ref[mb]

    @pl.when(j * CHUNK < va)
    def _():
        rows = jax.lax.broadcasted_iota(jnp.int32, (CHUNK, 1), 0) + j * CHUNK
        x = jnp.where(rows < va, nf_ref[0], jnp.float32(0.0))
        out_ref[...] += jnp.sum(x, axis=0)[None, None, :]

    @pl.when(j == MOLC - 1)
    def _():
        vf = vf_ref[mb]
        feat = jax.lax.broadcasted_iota(jnp.int32, (1, 1, FD), 2)
        res = jnp.where(feat < vf, out_ref[...], jnp.float32(0.0))
        out_ref[...] = jnp.maximum(res, jnp.float32(0.0))


@jax.jit
def _run(node_features, ds_flat):
    va = ds_flat[:B]
    vf = ds_flat[B:]
    last_chunk = jnp.maximum((va + CHUNK - 1) // CHUNK - 1, 0)

    grid_spec = pltpu.PrefetchScalarGridSpec(
        num_scalar_prefetch=3,
        grid=(TCB, MOLC),
        in_specs=[
            pl.BlockSpec(
                (1, CHUNK, FD),
                lambda b, j, lc, va, vf: (b + SCB, jnp.minimum(j, lc[b + SCB]), 0)),
        ],
        out_specs=pl.BlockSpec((1, 8, FD), lambda b, j, lc, va, vf: (b, 0, 0)),
    )
    out_tc = pl.pallas_call(
        _tc_mol_kernel,
        grid_spec=grid_spec,
        out_shape=jax.ShapeDtypeStruct((TCB, 8, FD), jnp.float32),
        compiler_params=pltpu.CompilerParams(
            dimension_semantics=("arbitrary", "arbitrary")),
    )(last_chunk, va, vf, node_features)
    out_tc = out_tc[:, 0, :]

    mesh = plsc.VectorSubcoreMesh(core_axis_name="c", subcore_axis_name="s")
    sc_fn = functools.partial(
        pl.kernel,
        mesh=mesh,
        out_type=jax.ShapeDtypeStruct((SCB, FD), jnp.float32),
        scratch_types=[
            pltpu.VMEM((CHUNK, FD), jnp.float32),
            pltpu.VMEM((CHUNK, FD), jnp.float32),
            pltpu.VMEM((CHUNK, FD), jnp.float32),
            pltpu.VMEM((2 * B + L,), jnp.int32),
            pltpu.VMEM((MAXSLOT, FD), jnp.float32),
            pltpu.VMEM((MOLC, FD), jnp.float32),
            pltpu.VMEM((FD,), jnp.float32),
            pltpu.VMEM_SHARED((PARTROWS + MOLC, FD), jnp.float32),
            pltpu.SemaphoreType.DMA,
            pltpu.SemaphoreType.DMA,
            pltpu.SemaphoreType.DMA,
            pltpu.SemaphoreType.DMA,
        ],
    )(_sc_mol_kernel)
    out_sc = sc_fn(node_features, ds_flat)

    return jnp.concatenate([out_sc, out_tc], axis=0)


def kernel(node_features, data_slice):
    ds = data_slice.astype(jnp.int32)
    ds_flat = jnp.concatenate([ds[:, 0], ds[:, 1]])
    return _run(node_features, ds_flat)


# speedup vs baseline: 1.5615x; 1.5615x over previous
import functools

import jax
import jax.numpy as jnp
from jax import lax
from jax.experimental import pallas as pl
from jax.experimental.pallas import tpu as pltpu
from jax.experimental.pallas import tpu_sc as plsc

B = 16
A = 4096
FD = 128
L = 16
NK = FD // L
CHUNK = 256
MOLC = A // CHUNK
SCB = 2
TCB = B - SCB
MPC = SCB // 2
NSUB = 16
MAXSLOT = max(1, MPC * MOLC // NSUB)
UNROLL = 8
PARTROWS = MPC * MOLC


def _sc_mol_kernel(nf_hbm, ds_hbm, out_hbm,
                   buf0, buf1, buf2, stage, rowp, comb, row_buf, shared,
                   sem0, sem1, sem2, sem_r):
    core = lax.axis_index("c")
    sub = lax.axis_index("s")
    idx16 = lax.iota(jnp.int32, L)

    pltpu.async_copy(ds_hbm, stage.at[pl.ds(0, 2 * B)], sem_r)
    pltpu.make_async_copy(ds_hbm, stage.at[pl.ds(0, 2 * B)], sem_r).wait()

    def va_of(j):
        return stage[pl.ds(MPC * core + j, L)][0]

    va_l = [va_of(j) for j in range(MPC)]
    cum = [jnp.int32(0)]
    for j in range(MPC):
        cum.append(cum[j] + (va_l[j] + CHUNK - 1) // CHUNK)
    total = cum[MPC]

    q = total // NSUB
    r = total % NSUB
    my_cnt = q + jnp.where(sub < r, 1, 0)
    my_start = sub * q + jnp.minimum(sub, r)

    def chunk_info(i):
        g = my_start + i
        lb = jnp.int32(0)
        ce = jnp.int32(0)
        va = va_l[0]
        for j in range(1, MPC):
            after = cum[j] <= g
            lb = lb + after.astype(jnp.int32)
            ce = jnp.where(after, cum[j], ce)
            va = jnp.where(after, va_l[j], va)
        jc = g - ce
        m = jnp.clip(va - jc * CHUNK, 0, CHUNK)
        return g, MPC * core + lb, jc, m

    bufs = [buf0, buf1, buf2]
    sems = [sem0, sem1, sem2]
    NBUF = 3

    def dma_start(i):
        g, b, jc, m = chunk_info(i)

        @pl.when(i < my_cnt)
        def _():
            pltpu.async_copy(
                nf_hbm.at[b, pl.ds(jc * CHUNK, CHUNK), :], bufs[i % NBUF],
                sems[i % NBUF])

    dma_start(0)
    if MAXSLOT > 1:
        dma_start(1)
    for i in range(MAXSLOT):
        if i + 2 < MAXSLOT:
            dma_start(i + 2)
        g, b, jc, m = chunk_info(i)
        buf = bufs[i % NBUF]

        @pl.when(i < my_cnt)
        def _(buf=buf, g=g, b=b, jc=jc, m=m, i=i):
            pltpu.make_async_copy(
                nf_hbm.at[b, pl.ds(jc * CHUNK, CHUNK), :], buf,
                sems[i % NBUF]).wait()
            accs = tuple(jnp.zeros((L,), jnp.float32) for _ in range(NK))

            def body(it, acc, buf=buf, m=m):
                base = it * UNROLL
                for rr in range(UNROLL):
                    j = base + rr
                    keep = j < m
                    acc = tuple(
                        acc[k] + jnp.where(keep, buf[j, pl.ds(k * L, L)],
                                           jnp.float32(0.0))
                        for k in range(NK))
                return acc

            ngroups = (m + UNROLL - 1) // UNROLL
            accs = lax.fori_loop(0, ngroups, body, accs)
            for k in range(NK):
                rowp[i, pl.ds(k * L, L)] = accs[k]
            pltpu.async_copy(rowp.at[i], shared.at[g], sem_r)

    for i in range(MAXSLOT):
        g, b, jc, m = chunk_info(i)

        @pl.when(i < my_cnt)
        def _(g=g, i=i):
            pltpu.make_async_copy(rowp.at[i], shared.at[g], sem_r).wait()
    plsc.subcore_barrier()

    @pl.when(sub < MPC)
    def _():
        b_out = MPC * core + sub
        cs = jnp.int32(0)
        n = jnp.int32(0)
        for j in range(MPC):
            here = sub == j
            cs = jnp.where(here, cum[j], cs)
            n = jnp.where(here, cum[j + 1] - cum[j], n)
        vf_b = stage[pl.ds(B + b_out, L)][0]
        pltpu.sync_copy(shared.at[pl.ds(cs, MOLC)], comb)
        accs = tuple(jnp.zeros((L,), jnp.float32) for _ in range(NK))
        for rr in range(MOLC):
            keep = rr < n
            accs = tuple(
                accs[k] + jnp.where(keep, comb[rr, pl.ds(k * L, L)],
                                    jnp.float32(0.0))
                for k in range(NK))
        for k in range(NK):
            fkeep = (idx16 + k * L) < vf_b
            row_buf[pl.ds(k * L, L)] = jnp.maximum(
                jnp.where(fkeep, accs[k], jnp.float32(0.0)), jnp.float32(0.0))
        pltpu.sync_copy(row_buf, out_hbm.at[b_out])


TCNBUF = 4
RG = CHUNK // 8


def _tc_mol_kernel(lc_ref, va_ref, vf_ref, nf_hbm, out_ref,
                   tb0, tb1, tb2, tb3, acc, ts0, ts1, ts2, ts3):
    b = pl.program_id(0)
    mb = b + SCB
    va = va_ref[mb]
    n = lc_ref[mb] + 1
    bufs = [tb0, tb1, tb2, tb3]
    sems = [ts0, ts1, ts2, ts3]

    def start(i):
        @pl.when(i < n)
        def _():
            pltpu.async_copy(nf_hbm.at[mb, pl.ds(i * CHUNK, CHUNK), :],
                             bufs[i % TCNBUF], sems[i % TCNBUF])

    start(0)
    start(1)
    start(2)
    acc[...] = jnp.zeros((8, FD), jnp.float32)
    for i in range(MOLC):
        if i + 3 < MOLC:
            start(i + 3)
        buf = bufs[i % TCNBUF]
        sem = sems[i % TCNBUF]

        @pl.when(i < n)
        def _(buf=buf, sem=sem, i=i):
            pltpu.make_async_copy(
                nf_hbm.at[mb, pl.ds(i * CHUNK, CHUNK), :], buf, sem).wait()
            x = buf[...].reshape(RG, 8, FD)

            @pl.when((i + 1) * CHUNK <= va)
            def _():
                acc[...] += jnp.sum(x, axis=0)

            @pl.when(((i + 1) * CHUNK > va) & (i * CHUNK < va))
            def _():
                ri = (jax.lax.broadcasted_iota(jnp.int32, (RG, 8, 1), 0) * 8
                      + jax.lax.broadcasted_iota(jnp.int32, (RG, 8, 1), 1)
                      + i * CHUNK)
                xm = jnp.where(ri < va, x, jnp.float32(0.0))
                acc[...] += jnp.sum(xm, axis=0)

    vf = vf_ref[mb]
    row = jnp.sum(acc[...], axis=0)
    feat = jax.lax.broadcasted_iota(jnp.int32, (FD,), 0)
    row = jnp.maximum(jnp.where(feat < vf, row, jnp.float32(0.0)),
                      jnp.float32(0.0))
    out_ref[...] = jnp.broadcast_to(row[None, None, :], (1, 8, FD))


@jax.jit
def _run(node_features, ds_flat):
    va = ds_flat[:B]
    vf = ds_flat[B:]
    last_chunk = jnp.maximum((va + CHUNK - 1) // CHUNK - 1, 0)

    grid_spec = pltpu.PrefetchScalarGridSpec(
        num_scalar_prefetch=3,
        grid=(TCB,),
        in_specs=[pl.BlockSpec(memory_space=pl.ANY)],
        out_specs=pl.BlockSpec((1, 8, FD), lambda b, lc, va, vf: (b, 0, 0)),
        scratch_shapes=[
            pltpu.VMEM((CHUNK, FD), jnp.float32),
            pltpu.VMEM((CHUNK, FD), jnp.float32),
            pltpu.VMEM((CHUNK, FD), jnp.float32),
            pltpu.VMEM((CHUNK, FD), jnp.float32),
            pltpu.VMEM((8, FD), jnp.float32),
            pltpu.SemaphoreType.DMA,
            pltpu.SemaphoreType.DMA,
            pltpu.SemaphoreType.DMA,
            pltpu.SemaphoreType.DMA,
        ],
    )
    out_tc = pl.pallas_call(
        _tc_mol_kernel,
        grid_spec=grid_spec,
        out_shape=jax.ShapeDtypeStruct((TCB, 8, FD), jnp.float32),
        compiler_params=pltpu.CompilerParams(
            dimension_semantics=("arbitrary",)),
    )(last_chunk, va, vf, node_features)
    out_tc = out_tc[:, 0, :]

    mesh = plsc.VectorSubcoreMesh(core_axis_name="c", subcore_axis_name="s")
    sc_fn = functools.partial(
        pl.kernel,
        mesh=mesh,
        out_type=jax.ShapeDtypeStruct((SCB, FD), jnp.float32),
        scratch_types=[
            pltpu.VMEM((CHUNK, FD), jnp.float32),
            pltpu.VMEM((CHUNK, FD), jnp.float32),
            pltpu.VMEM((CHUNK, FD), jnp.float32),
            pltpu.VMEM((2 * B + L,), jnp.int32),
            pltpu.VMEM((MAXSLOT, FD), jnp.float32),
            pltpu.VMEM((MOLC, FD), jnp.float32),
            pltpu.VMEM((FD,), jnp.float32),
            pltpu.VMEM_SHARED((PARTROWS + MOLC, FD), jnp.float32),
            pltpu.SemaphoreType.DMA,
            pltpu.SemaphoreType.DMA,
            pltpu.SemaphoreType.DMA,
            pltpu.SemaphoreType.DMA,
        ],
    )(_sc_mol_kernel)
    out_sc = sc_fn(node_features, ds_flat)

    return jnp.concatenate([out_sc, out_tc], axis=0)


def kernel(node_features, data_slice):
    ds = data_slice.astype(jnp.int32)
    ds_flat = jnp.concatenate([ds[:, 0], ds[:, 1]])
    return _run(node_features, ds_flat)


# speedup vs baseline: 1.5828x; 1.0136x over previous
import functools

import jax
import jax.numpy as jnp
from jax import lax
from jax.experimental import pallas as pl
from jax.experimental.pallas import tpu as pltpu
from jax.experimental.pallas import tpu_sc as plsc

B = 16
A = 4096
FD = 128
L = 16
NK = FD // L
CHUNK = 256
MOLC = A // CHUNK
SCB = 2
TCB = B - SCB
MPC = SCB // 2
NSUB = 16
MAXSLOT = max(1, MPC * MOLC // NSUB)
UNROLL = 8
PARTROWS = MPC * MOLC


def _sc_mol_kernel(nf_hbm, ds_hbm, out_hbm,
                   buf0, buf1, buf2, stage, rowp, comb, row_buf, shared,
                   sem0, sem1, sem2, sem_r):
    core = lax.axis_index("c")
    sub = lax.axis_index("s")
    idx16 = lax.iota(jnp.int32, L)

    pltpu.async_copy(ds_hbm, stage.at[pl.ds(0, 2 * B)], sem_r)
    pltpu.make_async_copy(ds_hbm, stage.at[pl.ds(0, 2 * B)], sem_r).wait()

    def va_of(j):
        return stage[pl.ds(MPC * core + j, L)][0]

    va_l = [va_of(j) for j in range(MPC)]
    cum = [jnp.int32(0)]
    for j in range(MPC):
        cum.append(cum[j] + (va_l[j] + CHUNK - 1) // CHUNK)
    total = cum[MPC]

    q = total // NSUB
    r = total % NSUB
    my_cnt = q + jnp.where(sub < r, 1, 0)
    my_start = sub * q + jnp.minimum(sub, r)

    def chunk_info(i):
        g = my_start + i
        lb = jnp.int32(0)
        ce = jnp.int32(0)
        va = va_l[0]
        for j in range(1, MPC):
            after = cum[j] <= g
            lb = lb + after.astype(jnp.int32)
            ce = jnp.where(after, cum[j], ce)
            va = jnp.where(after, va_l[j], va)
        jc = g - ce
        m = jnp.clip(va - jc * CHUNK, 0, CHUNK)
        return g, MPC * core + lb, jc, m

    bufs = [buf0, buf1, buf2]
    sems = [sem0, sem1, sem2]
    NBUF = 3

    def dma_start(i):
        g, b, jc, m = chunk_info(i)

        @pl.when(i < my_cnt)
        def _():
            pltpu.async_copy(
                nf_hbm.at[b, pl.ds(jc * CHUNK, CHUNK), :], bufs[i % NBUF],
                sems[i % NBUF])

    dma_start(0)
    if MAXSLOT > 1:
        dma_start(1)
    for i in range(MAXSLOT):
        if i + 2 < MAXSLOT:
            dma_start(i + 2)
        g, b, jc, m = chunk_info(i)
        buf = bufs[i % NBUF]

        @pl.when(i < my_cnt)
        def _(buf=buf, g=g, b=b, jc=jc, m=m, i=i):
            pltpu.make_async_copy(
                nf_hbm.at[b, pl.ds(jc * CHUNK, CHUNK), :], buf,
                sems[i % NBUF]).wait()
            accs = tuple(jnp.zeros((L,), jnp.float32) for _ in range(NK))

            def body(it, acc, buf=buf, m=m):
                base = it * UNROLL
                for rr in range(UNROLL):
                    j = base + rr
                    keep = j < m
                    acc = tuple(
                        acc[k] + jnp.where(keep, buf[j, pl.ds(k * L, L)],
                                           jnp.float32(0.0))
                        for k in range(NK))
                return acc

            ngroups = (m + UNROLL - 1) // UNROLL
            accs = lax.fori_loop(0, ngroups, body, accs)
            for k in range(NK):
                rowp[i, pl.ds(k * L, L)] = accs[k]
            pltpu.async_copy(rowp.at[i], shared.at[g], sem_r)

    for i in range(MAXSLOT):
        g, b, jc, m = chunk_info(i)

        @pl.when(i < my_cnt)
        def _(g=g, i=i):
            pltpu.make_async_copy(rowp.at[i], shared.at[g], sem_r).wait()
    plsc.subcore_barrier()

    @pl.when(sub < MPC)
    def _():
        b_out = MPC * core + sub
        cs = jnp.int32(0)
        n = jnp.int32(0)
        for j in range(MPC):
            here = sub == j
            cs = jnp.where(here, cum[j], cs)
            n = jnp.where(here, cum[j + 1] - cum[j], n)
        vf_b = stage[pl.ds(B + b_out, L)][0]
        pltpu.sync_copy(shared.at[pl.ds(cs, MOLC)], comb)
        accs = tuple(jnp.zeros((L,), jnp.float32) for _ in range(NK))
        for rr in range(MOLC):
            keep = rr < n
            accs = tuple(
                accs[k] + jnp.where(keep, comb[rr, pl.ds(k * L, L)],
                                    jnp.float32(0.0))
                for k in range(NK))
        for k in range(NK):
            fkeep = (idx16 + k * L) < vf_b
            row_buf[pl.ds(k * L, L)] = jnp.maximum(
                jnp.where(fkeep, accs[k], jnp.float32(0.0)), jnp.float32(0.0))
        pltpu.sync_copy(row_buf, out_hbm.at[b_out])


TCNBUF = 4
RG = CHUNK // 8


def _tc_mol_kernel(lc_ref, va_ref, vf_ref, nf_hbm, out_ref,
                   tb0, tb1, tb2, tb3, acc, ts0, ts1, ts2, ts3):
    b = pl.program_id(0)
    mb = b + SCB
    va = va_ref[mb]
    n = lc_ref[mb] + 1
    bufs = [tb0, tb1, tb2, tb3]
    sems = [ts0, ts1, ts2, ts3]

    def start(i):
        @pl.when(i < n)
        def _():
            pltpu.async_copy(nf_hbm.at[mb, pl.ds(i * CHUNK, CHUNK), :],
                             bufs[i % TCNBUF], sems[i % TCNBUF])

    start(0)
    start(1)
    start(2)
    acc[...] = jnp.zeros((CHUNK, FD), jnp.float32)
    for i in range(MOLC):
        if i + 3 < MOLC:
            start(i + 3)
        buf = bufs[i % TCNBUF]
        sem = sems[i % TCNBUF]

        @pl.when(i < n)
        def _(buf=buf, sem=sem, i=i):
            pltpu.make_async_copy(
                nf_hbm.at[mb, pl.ds(i * CHUNK, CHUNK), :], buf, sem).wait()

            @pl.when((i + 1) * CHUNK <= va)
            def _():
                acc[...] += buf[...]

            @pl.when(((i + 1) * CHUNK > va) & (i * CHUNK < va))
            def _():
                ri = (jax.lax.broadcasted_iota(jnp.int32, (CHUNK, 1), 0)
                      + i * CHUNK)
                acc[...] += jnp.where(ri < va, buf[...], jnp.float32(0.0))

    vf = vf_ref[mb]
    row = jnp.sum(acc[...], axis=0)
    feat = jax.lax.broadcasted_iota(jnp.int32, (FD,), 0)
    row = jnp.maximum(jnp.where(feat < vf, row, jnp.float32(0.0)),
                      jnp.float32(0.0))
    out_ref[...] = jnp.broadcast_to(row[None, None, :], (1, 8, FD))


@jax.jit
def _run(node_features, ds_flat):
    va = ds_flat[:B]
    vf = ds_flat[B:]
    last_chunk = jnp.maximum((va + CHUNK - 1) // CHUNK - 1, 0)

    grid_spec = pltpu.PrefetchScalarGridSpec(
        num_scalar_prefetch=3,
        grid=(TCB,),
        in_specs=[pl.BlockSpec(memory_space=pl.ANY)],
        out_specs=pl.BlockSpec((1, 8, FD), lambda b, lc, va, vf: (b, 0, 0)),
        scratch_shapes=[
            pltpu.VMEM((CHUNK, FD), jnp.float32),
            pltpu.VMEM((CHUNK, FD), jnp.float32),
            pltpu.VMEM((CHUNK, FD), jnp.float32),
            pltpu.VMEM((CHUNK, FD), jnp.float32),
            pltpu.VMEM((CHUNK, FD), jnp.float32),
            pltpu.SemaphoreType.DMA,
            pltpu.SemaphoreType.DMA,
            pltpu.SemaphoreType.DMA,
            pltpu.SemaphoreType.DMA,
        ],
    )
    out_tc = pl.pallas_call(
        _tc_mol_kernel,
        grid_spec=grid_spec,
        out_shape=jax.ShapeDtypeStruct((TCB, 8, FD), jnp.float32),
        compiler_params=pltpu.CompilerParams(
            dimension_semantics=("arbitrary",)),
    )(last_chunk, va, vf, node_features)
    out_tc = out_tc[:, 0, :]

    mesh = plsc.VectorSubcoreMesh(core_axis_name="c", subcore_axis_name="s")
    sc_fn = functools.partial(
        pl.kernel,
        mesh=mesh,
        out_type=jax.ShapeDtypeStruct((SCB, FD), jnp.float32),
        scratch_types=[
            pltpu.VMEM((CHUNK, FD), jnp.float32),
            pltpu.VMEM((CHUNK, FD), jnp.float32),
            pltpu.VMEM((CHUNK, FD), jnp.float32),
            pltpu.VMEM((2 * B + L,), jnp.int32),
            pltpu.VMEM((MAXSLOT, FD), jnp.float32),
            pltpu.VMEM((MOLC, FD), jnp.float32),
            pltpu.VMEM((FD,), jnp.float32),
            pltpu.VMEM_SHARED((PARTROWS + MOLC, FD), jnp.float32),
            pltpu.SemaphoreType.DMA,
            pltpu.SemaphoreType.DMA,
            pltpu.SemaphoreType.DMA,
            pltpu.SemaphoreType.DMA,
        ],
    )(_sc_mol_kernel)
    out_sc = sc_fn(node_features, ds_flat)

    return jnp.concatenate([out_sc, out_tc], axis=0)


def kernel(node_features, data_slice):
    ds = data_slice.astype(jnp.int32)
    ds_flat = jnp.concatenate([ds[:, 0], ds[:, 1]])
    return _run(node_features, ds_flat)
